# baseline (speedup 1.0000x reference)
import math

import jax
import jax.numpy as jnp
from jax.experimental import pallas as pl
from jax.experimental.pallas import tpu as pltpu

_B, _S, _H, _E = 4, 2048, 1024, 16
_ROWS = _B * _S
_NOPS = 8
_GRID = 4
_CHUNK = _ROWS // (_NOPS * _GRID)
_OPS_PER_BATCH = _NOPS // _B
_INV_SQRT2 = 1.0 / math.sqrt(2.0)


def _gate_kernel(*refs):
    xs = refs[:_NOPS]
    w1_ref, b1_ref, w2_ref, b2_ref = refs[_NOPS:_NOPS + 4]
    tw_ref, ti_ref = refs[_NOPS + 4:_NOPS + 6]
    accs = refs[_NOPS + 6:]
    step = pl.program_id(0)

    @pl.when(step == 0)
    def _init():
        for acc, x in zip(accs, xs):
            acc[...] = x[...]

    @pl.when(step != 0)
    def _accum():
        for acc, x in zip(accs, xs):
            acc[...] += x[...]

    @pl.when(step == _GRID - 1)
    def _tail():
        rows = []
        for b in range(_B):
            t = accs[_OPS_PER_BATCH * b][...]
            for j in range(1, _OPS_PER_BATCH):
                t = t + accs[_OPS_PER_BATCH * b + j][...]
            rows.append(jnp.sum(t, axis=0, keepdims=True))
        seq = jnp.concatenate(rows, axis=0) * (1.0 / _S)
        h = jnp.dot(seq, w1_ref[...], preferred_element_type=jnp.float32)
        h = h + b1_ref[...]
        h = 0.5 * h * (1.0 + jax.lax.erf(h * _INV_SQRT2))
        logits = jnp.dot(h, w2_ref[...], preferred_element_type=jnp.float32)
        logits = logits + b2_ref[...]
        m = jnp.max(logits, axis=1, keepdims=True)
        ex = jnp.exp(logits - m)
        probs = ex / jnp.sum(ex, axis=1, keepdims=True)
        lane = jax.lax.broadcasted_iota(jnp.int32, probs.shape, 1)
        p1 = jnp.max(probs, axis=1, keepdims=True)
        i1 = jnp.min(jnp.where(probs == p1, lane, _E), axis=1, keepdims=True)
        masked = jnp.where(lane == i1, -1.0, probs)
        p2 = jnp.max(masked, axis=1, keepdims=True)
        i2 = jnp.min(jnp.where(masked == p2, lane, _E), axis=1, keepdims=True)
        e2 = jnp.exp(p2 - p1)
        denom = 1.0 + e2
        tw_ref[...] = jnp.concatenate([1.0 / denom, e2 / denom], axis=1)
        ti_ref[...] = jnp.concatenate([i1, i2], axis=1)


def _x_spec(op_idx):
    return pl.BlockSpec((_CHUNK, _H),
                        lambda i, j=op_idx: (_GRID * j + i, 0))


def kernel(hidden_states, W1, b1, W2, b2):
    hs2 = hidden_states.reshape(_ROWS, _H)
    tw, ti = pl.pallas_call(
        _gate_kernel,
        grid=(_GRID,),
        in_specs=(
            [_x_spec(j) for j in range(_NOPS)]
            + [
                pl.BlockSpec((_H, _H), lambda i: (0, 0)),
                pl.BlockSpec((_H,), lambda i: (0,)),
                pl.BlockSpec((_H, _E), lambda i: (0, 0)),
                pl.BlockSpec((_E,), lambda i: (0,)),
            ]
        ),
        out_specs=[
            pl.BlockSpec((_B, 2), lambda i: (0, 0)),
            pl.BlockSpec((_B, 2), lambda i: (0, 0)),
        ],
        out_shape=[
            jax.ShapeDtypeStruct((_B, 2), jnp.float32),
            jax.ShapeDtypeStruct((_B, 2), jnp.int32),
        ],
        scratch_shapes=[pltpu.VMEM((_CHUNK, _H), jnp.float32) for _ in range(_NOPS)],
    )(*([hs2] * _NOPS), W1, b1, W2, b2)
    return tw, ti


# speedup vs baseline: 1.0197x; 1.0197x over previous
import math

import jax
import jax.numpy as jnp
from jax.experimental import pallas as pl
from jax.experimental.pallas import tpu as pltpu

_B, _S, _H, _E = 4, 2048, 1024, 16
_ROWS = _B * _S
_CHUNK = 256
_NCHUNKS = _ROWS // _CHUNK
_CHUNKS_PER_BATCH = _NCHUNKS // _B
_INV_SQRT2 = 1.0 / math.sqrt(2.0)


def _fold8(c):
    t = c[0:128] + c[128:256]
    t = t[0:64] + t[64:128]
    t = t[0:32] + t[32:64]
    t = t[0:16] + t[16:32]
    return t[0:8] + t[8:16]


def _gate_kernel(x_ref, w1_ref, b1_ref, w2_ref, b2_ref, tw_ref, ti_ref,
                 buf_ref, sem):
    copies = [
        pltpu.make_async_copy(
            x_ref.at[pl.ds(_CHUNK * i, _CHUNK), :], buf_ref.at[i], sem.at[i])
        for i in range(_NCHUNKS)
    ]
    for c in copies:
        c.start()

    batch_sums = []
    for b in range(_B):
        acc = None
        for j in range(_CHUNKS_PER_BATCH):
            i = _CHUNKS_PER_BATCH * b + j
            copies[i].wait()
            f = _fold8(buf_ref[i])
            acc = f if acc is None else acc + f
        batch_sums.append(jnp.sum(acc, axis=0, keepdims=True))

    seq = jnp.concatenate(batch_sums, axis=0) * (1.0 / _S)
    h = jnp.dot(seq, w1_ref[...], preferred_element_type=jnp.float32)
    h = h + b1_ref[...]
    h = 0.5 * h * (1.0 + jax.lax.erf(h * _INV_SQRT2))
    logits = jnp.dot(h, w2_ref[...], preferred_element_type=jnp.float32)
    logits = logits + b2_ref[...]
    m = jnp.max(logits, axis=1, keepdims=True)
    ex = jnp.exp(logits - m)
    probs = ex / jnp.sum(ex, axis=1, keepdims=True)
    lane = jax.lax.broadcasted_iota(jnp.int32, probs.shape, 1)
    p1 = jnp.max(probs, axis=1, keepdims=True)
    i1 = jnp.min(jnp.where(probs == p1, lane, _E), axis=1, keepdims=True)
    masked = jnp.where(lane == i1, -1.0, probs)
    p2 = jnp.max(masked, axis=1, keepdims=True)
    i2 = jnp.min(jnp.where(masked == p2, lane, _E), axis=1, keepdims=True)
    e2 = jnp.exp(p2 - p1)
    denom = 1.0 + e2
    tw_ref[...] = jnp.concatenate([1.0 / denom, e2 / denom], axis=1)
    ti_ref[...] = jnp.concatenate([i1, i2], axis=1)


def kernel(hidden_states, W1, b1, W2, b2):
    hs2 = hidden_states.reshape(_ROWS, _H)
    tw, ti = pl.pallas_call(
        _gate_kernel,
        in_specs=[
            pl.BlockSpec(memory_space=pltpu.MemorySpace.HBM),
            pl.BlockSpec((_H, _H), lambda: (0, 0)),
            pl.BlockSpec((_H,), lambda: (0,)),
            pl.BlockSpec((_H, _E), lambda: (0, 0)),
            pl.BlockSpec((_E,), lambda: (0,)),
        ],
        out_specs=[
            pl.BlockSpec((_B, 2), lambda: (0, 0)),
            pl.BlockSpec((_B, 2), lambda: (0, 0)),
        ],
        out_shape=[
            jax.ShapeDtypeStruct((_B, 2), jnp.float32),
            jax.ShapeDtypeStruct((_B, 2), jnp.int32),
        ],
        scratch_shapes=[
            pltpu.VMEM((_NCHUNKS, _CHUNK, _H), jnp.float32),
            pltpu.SemaphoreType.DMA((_NCHUNKS,)),
        ],
    )(hs2, W1, b1, W2, b2)
    return tw, ti


# speedup vs baseline: 1.0366x; 1.0166x over previous
import math

import jax
import jax.numpy as jnp
from jax.experimental import pallas as pl
from jax.experimental.pallas import tpu as pltpu

_B, _S, _H, _E = 4, 2048, 1024, 16
_ROWS = _B * _S
_CHUNK = 256
_NCHUNKS = _ROWS // _CHUNK
_CHUNKS_PER_BATCH = _NCHUNKS // _B
_INV_SQRT2 = 1.0 / math.sqrt(2.0)


def _fold8(c):
    t = c[0:128] + c[128:256]
    t = t[0:64] + t[64:128]
    t = t[0:32] + t[32:64]
    t = t[0:16] + t[16:32]
    return t[0:8] + t[8:16]


def _gate_kernel(x_ref, w1_ref, b1_ref, w2_ref, b2_ref, tw_ref, ti_ref,
                 buf_ref, sem):
    copies = [
        pltpu.make_async_copy(
            x_ref.at[pl.ds(_CHUNK * i, _CHUNK), :], buf_ref.at[i], sem.at[i])
        for i in range(_NCHUNKS)
    ]
    for i, c in enumerate(copies):
        c.start(priority=i % 2)

    batch_sums = []
    for b in range(_B):
        acc = None
        for j in range(_CHUNKS_PER_BATCH):
            i = _CHUNKS_PER_BATCH * b + j
            copies[i].wait()
            f = _fold8(buf_ref[i])
            acc = f if acc is None else acc + f
        batch_sums.append(jnp.sum(acc, axis=0, keepdims=True))

    seq = jnp.concatenate(batch_sums, axis=0) * (1.0 / _S)
    h = jnp.dot(seq, w1_ref[...], preferred_element_type=jnp.float32)
    h = h + b1_ref[...]
    h = 0.5 * h * (1.0 + jax.lax.erf(h * _INV_SQRT2))
    logits = jnp.dot(h, w2_ref[...], preferred_element_type=jnp.float32)
    logits = logits + b2_ref[...]
    m = jnp.max(logits, axis=1, keepdims=True)
    ex = jnp.exp(logits - m)
    probs = ex / jnp.sum(ex, axis=1, keepdims=True)
    lane = jax.lax.broadcasted_iota(jnp.int32, probs.shape, 1)
    p1 = jnp.max(probs, axis=1, keepdims=True)
    i1 = jnp.min(jnp.where(probs == p1, lane, _E), axis=1, keepdims=True)
    masked = jnp.where(lane == i1, -1.0, probs)
    p2 = jnp.max(masked, axis=1, keepdims=True)
    i2 = jnp.min(jnp.where(masked == p2, lane, _E), axis=1, keepdims=True)
    e2 = jnp.exp(p2 - p1)
    denom = 1.0 + e2
    tw_ref[...] = jnp.concatenate([1.0 / denom, e2 / denom], axis=1)
    ti_ref[...] = jnp.concatenate([i1, i2], axis=1)


def kernel(hidden_states, W1, b1, W2, b2):
    hs2 = hidden_states.reshape(_ROWS, _H)
    tw, ti = pl.pallas_call(
        _gate_kernel,
        in_specs=[
            pl.BlockSpec(memory_space=pltpu.MemorySpace.HBM),
            pl.BlockSpec((_H, _H), lambda: (0, 0)),
            pl.BlockSpec((_H,), lambda: (0,)),
            pl.BlockSpec((_H, _E), lambda: (0, 0)),
            pl.BlockSpec((_E,), lambda: (0,)),
        ],
        out_specs=[
            pl.BlockSpec((_B, 2), lambda: (0, 0)),
            pl.BlockSpec((_B, 2), lambda: (0, 0)),
        ],
        out_shape=[
            jax.ShapeDtypeStruct((_B, 2), jnp.float32),
            jax.ShapeDtypeStruct((_B, 2), jnp.int32),
        ],
        scratch_shapes=[
            pltpu.VMEM((_NCHUNKS, _CHUNK, _H), jnp.float32),
            pltpu.SemaphoreType.DMA((_NCHUNKS,)),
        ],
    )(hs2, W1, b1, W2, b2)
    return tw, ti


# speedup vs baseline: 1.0387x; 1.0020x over previous
import math

import jax
import jax.numpy as jnp
from jax.experimental import pallas as pl
from jax.experimental.pallas import tpu as pltpu

_B, _S, _H, _E = 4, 2048, 1024, 16
_ROWS = _B * _S
_CHUNK = 256
_NCHUNKS = _ROWS // _CHUNK
_CHUNKS_PER_BATCH = _NCHUNKS // _B
_STRIP = 32
_INV_SQRT2 = 1.0 / math.sqrt(2.0)


def _chunk_sum8(buf_ref, i):
    acc = None
    for s in range(_CHUNK // _STRIP):
        k = _STRIP * s
        t16 = buf_ref[i, k:k + 16, :] + buf_ref[i, k + 16:k + 32, :]
        t8 = t16[0:8] + t16[8:16]
        acc = t8 if acc is None else acc + t8
    return acc


def _gate_kernel(x_ref, w1_ref, b1_ref, w2_ref, b2_ref, tw_ref, ti_ref,
                 buf_ref, sem):
    copies = [
        pltpu.make_async_copy(
            x_ref.at[pl.ds(_CHUNK * i, _CHUNK), :], buf_ref.at[i], sem.at[i])
        for i in range(_NCHUNKS)
    ]
    for i, c in enumerate(copies):
        c.start(priority=i % 2)

    batch_sums = []
    for b in range(_B):
        acc = None
        for j in range(_CHUNKS_PER_BATCH):
            i = _CHUNKS_PER_BATCH * b + j
            copies[i].wait()
            f = _chunk_sum8(buf_ref, i)
            acc = f if acc is None else acc + f
        batch_sums.append(jnp.sum(acc, axis=0, keepdims=True))

    seq = jnp.concatenate(batch_sums, axis=0) * (1.0 / _S)
    h = jnp.dot(seq, w1_ref[...], preferred_element_type=jnp.float32)
    h = h + b1_ref[...]
    h = 0.5 * h * (1.0 + jax.lax.erf(h * _INV_SQRT2))
    logits = jnp.dot(h, w2_ref[...], preferred_element_type=jnp.float32)
    logits = logits + b2_ref[...]
    m = jnp.max(logits, axis=1, keepdims=True)
    ex = jnp.exp(logits - m)
    probs = ex / jnp.sum(ex, axis=1, keepdims=True)
    lane = jax.lax.broadcasted_iota(jnp.int32, probs.shape, 1)
    p1 = jnp.max(probs, axis=1, keepdims=True)
    i1 = jnp.min(jnp.where(probs == p1, lane, _E), axis=1, keepdims=True)
    masked = jnp.where(lane == i1, -1.0, probs)
    p2 = jnp.max(masked, axis=1, keepdims=True)
    i2 = jnp.min(jnp.where(masked == p2, lane, _E), axis=1, keepdims=True)
    e2 = jnp.exp(p2 - p1)
    denom = 1.0 + e2
    tw_ref[...] = jnp.concatenate([1.0 / denom, e2 / denom], axis=1)
    ti_ref[...] = jnp.concatenate([i1, i2], axis=1)


def kernel(hidden_states, W1, b1, W2, b2):
    hs2 = hidden_states.reshape(_ROWS, _H)
    tw, ti = pl.pallas_call(
        _gate_kernel,
        in_specs=[
            pl.BlockSpec(memory_space=pltpu.MemorySpace.HBM),
            pl.BlockSpec((_H, _H), lambda: (0, 0)),
            pl.BlockSpec((_H,), lambda: (0,)),
            pl.BlockSpec((_H, _E), lambda: (0, 0)),
            pl.BlockSpec((_E,), lambda: (0,)),
        ],
        out_specs=[
            pl.BlockSpec((_B, 2), lambda: (0, 0)),
            pl.BlockSpec((_B, 2), lambda: (0, 0)),
        ],
        out_shape=[
            jax.ShapeDtypeStruct((_B, 2), jnp.float32),
            jax.ShapeDtypeStruct((_B, 2), jnp.int32),
        ],
        scratch_shapes=[
            pltpu.VMEM((_NCHUNKS, _CHUNK, _H), jnp.float32),
            pltpu.SemaphoreType.DMA((_NCHUNKS,)),
        ],
    )(hs2, W1, b1, W2, b2)
    return tw, ti


# speedup vs baseline: 1.0442x; 1.0053x over previous
import math

import jax
import jax.numpy as jnp
from jax.experimental import pallas as pl
from jax.experimental.pallas import tpu as pltpu

_B, _S, _H, _E = 4, 2048, 1024, 16
_ROWS = _B * _S
_CHUNK = 512
_NCHUNKS = _ROWS // _CHUNK
_CHUNKS_PER_BATCH = _NCHUNKS // _B
_STRIP = 32
_INV_SQRT2 = 1.0 / math.sqrt(2.0)


def _chunk_sum8(buf_ref, i):
    acc = None
    for s in range(_CHUNK // _STRIP):
        k = _STRIP * s
        t16 = buf_ref[i, k:k + 16, :] + buf_ref[i, k + 16:k + 32, :]
        t8 = t16[0:8] + t16[8:16]
        acc = t8 if acc is None else acc + t8
    return acc


def _gate_kernel(x_ref, w1_ref, b1_ref, w2_ref, b2_ref, tw_ref, ti_ref,
                 buf_ref, sem):
    copies = [
        pltpu.make_async_copy(
            x_ref.at[pl.ds(_CHUNK * i, _CHUNK), :], buf_ref.at[i], sem.at[i])
        for i in range(_NCHUNKS)
    ]
    for i, c in enumerate(copies):
        c.start(priority=i % 2)

    batch_sums = []
    for b in range(_B):
        acc = None
        for j in range(_CHUNKS_PER_BATCH):
            i = _CHUNKS_PER_BATCH * b + j
            copies[i].wait()
            f = _chunk_sum8(buf_ref, i)
            acc = f if acc is None else acc + f
        batch_sums.append(jnp.sum(acc, axis=0, keepdims=True))

    seq = jnp.concatenate(batch_sums, axis=0) * (1.0 / _S)
    h = jnp.dot(seq, w1_ref[...], preferred_element_type=jnp.float32)
    h = h + b1_ref[...]
    h = 0.5 * h * (1.0 + jax.lax.erf(h * _INV_SQRT2))
    logits = jnp.dot(h, w2_ref[...], preferred_element_type=jnp.float32)
    logits = logits + b2_ref[...]
    m = jnp.max(logits, axis=1, keepdims=True)
    ex = jnp.exp(logits - m)
    probs = ex / jnp.sum(ex, axis=1, keepdims=True)
    lane = jax.lax.broadcasted_iota(jnp.int32, probs.shape, 1)
    p1 = jnp.max(probs, axis=1, keepdims=True)
    i1 = jnp.min(jnp.where(probs == p1, lane, _E), axis=1, keepdims=True)
    masked = jnp.where(lane == i1, -1.0, probs)
    p2 = jnp.max(masked, axis=1, keepdims=True)
    i2 = jnp.min(jnp.where(masked == p2, lane, _E), axis=1, keepdims=True)
    e2 = jnp.exp(p2 - p1)
    denom = 1.0 + e2
    tw_ref[...] = jnp.concatenate([1.0 / denom, e2 / denom], axis=1)
    ti_ref[...] = jnp.concatenate([i1, i2], axis=1)


def kernel(hidden_states, W1, b1, W2, b2):
    hs2 = hidden_states.reshape(_ROWS, _H)
    tw, ti = pl.pallas_call(
        _gate_kernel,
        in_specs=[
            pl.BlockSpec(memory_space=pltpu.MemorySpace.HBM),
            pl.BlockSpec((_H, _H), lambda: (0, 0)),
            pl.BlockSpec((_H,), lambda: (0,)),
            pl.BlockSpec((_H, _E), lambda: (0, 0)),
            pl.BlockSpec((_E,), lambda: (0,)),
        ],
        out_specs=[
            pl.BlockSpec((_B, 2), lambda: (0, 0)),
            pl.BlockSpec((_B, 2), lambda: (0, 0)),
        ],
        out_shape=[
            jax.ShapeDtypeStruct((_B, 2), jnp.float32),
            jax.ShapeDtypeStruct((_B, 2), jnp.int32),
        ],
        scratch_shapes=[
            pltpu.VMEM((_NCHUNKS, _CHUNK, _H), jnp.float32),
            pltpu.SemaphoreType.DMA((_NCHUNKS,)),
        ],
    )(hs2, W1, b1, W2, b2)
    return tw, ti
